# baseline (speedup 1.0000x reference)
import functools

import jax
import jax.numpy as jnp
from jax.experimental import pallas as pl


HIGH = jax.lax.Precision.DEFAULT
LOW = jax.lax.Precision.DEFAULT


def _conv_body(x_ref, w_ref, b_ref, o_ref, *, taps, H, W, C, oc, relu,
               precision, phased):
    if phased:
        p = pl.program_id(1)
        r, s = p // 2, p % 2
        parts = [x_ref[pl.ds(r + a, H), pl.ds(s + b, W), :]
                 for (a, b) in taps]
    else:
        parts = [x_ref[a:a + H, b:b + W, :] for (a, b) in taps]
    xc = jnp.concatenate(parts, axis=-1).reshape(H * W, len(taps) * C)
    acc = jax.lax.dot_general(
        xc, w_ref[0], (((1,), (0,)), ((), ())),
        preferred_element_type=jnp.float32, precision=precision)
    acc = acc + b_ref[0, 0]
    if relu:
        acc = jnp.maximum(acc, 0.0)
    o_ref[...] = acc.reshape(H, W, oc)


def _shifted_conv(xpad, wstk, bias, taps, H, W, relu, precision,
                  phased=False):
    B, Hp, Wp, C = xpad.shape
    P, K, oc = wstk.shape
    body = functools.partial(_conv_body, taps=taps, H=H, W=W,
                             C=C, oc=oc, relu=relu, precision=precision,
                             phased=phased)
    return pl.pallas_call(
        body,
        grid=(B, P),
        in_specs=[
            pl.BlockSpec((None, Hp, Wp, C), lambda b, p: (b, 0, 0, 0)),
            pl.BlockSpec((1, K, oc), lambda b, p: (p, 0, 0)),
            pl.BlockSpec((1, 1, oc), lambda b, p: (p, 0, 0)),
        ],
        out_specs=pl.BlockSpec((None, None, H, W, oc),
                               lambda b, p: (b, p, 0, 0, 0)),
        out_shape=jax.ShapeDtypeStruct((B, P, H, W, oc), jnp.float32),
    )(xpad, wstk, bias)


def _vq_body(z_ref, cbt_ref, cb_ref, q_ref):
    z = z_ref[...]
    cbt = cbt_ref[...]
    scores = jax.lax.dot_general(
        z, cbt, (((1,), (0,)), ((), ())),
        preferred_element_type=jnp.float32, precision=HIGH)
    cb2 = jnp.sum(cbt * cbt, axis=0, keepdims=True)
    d2 = cb2 - 2.0 * scores
    dmin = jnp.min(d2, axis=1, keepdims=True)
    iota = jax.lax.broadcasted_iota(jnp.int32, d2.shape, 1)
    masked = jnp.where(d2 <= dmin, iota, d2.shape[1])
    idx = jnp.min(masked, axis=1, keepdims=True)
    onehot = (iota == idx).astype(jnp.float32)
    q = jax.lax.dot_general(
        onehot, cb_ref[...], (((1,), (0,)), ((), ())),
        preferred_element_type=jnp.float32, precision=HIGH)
    q_ref[...] = q


def _vq(zf, cb, blocks=16):
    N, D = zf.shape
    K = cb.shape[0]
    M = N // blocks
    return pl.pallas_call(
        _vq_body,
        grid=(blocks,),
        in_specs=[
            pl.BlockSpec((M, D), lambda b: (b, 0)),
            pl.BlockSpec((D, K), lambda b: (0, 0)),
            pl.BlockSpec((K, D), lambda b: (0, 0)),
        ],
        out_specs=pl.BlockSpec((M, D), lambda b: (b, 0)),
        out_shape=jax.ShapeDtypeStruct((N, D), jnp.float32),
    )(zf, cb.T, cb)


def _s2d(x):
    B, H, W, C = x.shape
    x = x.reshape(B, H // 2, 2, W // 2, 2, C)
    x = x.transpose(0, 1, 3, 2, 4, 5)
    return x.reshape(B, H // 2, W // 2, 4 * C)


def _w_s2d(w):
    O, C = w.shape[0], w.shape[1]
    wt = w.transpose(2, 3, 1, 0)
    wt = wt.reshape(2, 2, 2, 2, C, O)
    wt = wt.transpose(0, 2, 1, 3, 4, 5)
    return wt.reshape(1, 4 * 4 * C, O)


def _w_3x3(w):
    O, C = w.shape[0], w.shape[1]
    return w.transpose(2, 3, 1, 0).reshape(1, 9 * C, O)


def _w_t_phases(w, opad=None):
    O, I = w.shape[0], w.shape[1]
    wt = w.transpose(2, 3, 1, 0)
    if opad is not None and opad > O:
        wt = jnp.pad(wt, ((0, 0), (0, 0), (0, 0), (0, opad - O)))
        O = opad
    phases = []
    for r in (0, 1):
        for s in (0, 1):
            ph = wt[r::2, s::2]
            phases.append(ph.reshape(4 * I, O))
    return jnp.stack(phases)


_TAPS2 = [(a, b) for a in (0, 1) for b in (0, 1)]
_TAPS3 = [(a, b) for a in (0, 1, 2) for b in (0, 1, 2)]


def kernel(x, W1, b1, W2, b2, W3, b3, codebook, Wd1, bd1, Wd2, bd2, Wd3, bd3):
    B = x.shape[0]

    xh = x.transpose(0, 2, 3, 1)
    xh = jnp.pad(xh, ((0, 0), (1, 1), (1, 1), (0, 0)))
    xh = _s2d(xh)
    h = _shifted_conv(xh, _w_s2d(W1), b1.reshape(1, 1, -1), _TAPS2,
                      112, 112, relu=True, precision=HIGH)[:, 0]

    h = jnp.pad(h, ((0, 0), (1, 1), (1, 1), (0, 0)))
    h = _s2d(h)
    h = _shifted_conv(h, _w_s2d(W2), b2.reshape(1, 1, -1), _TAPS2,
                      56, 56, relu=True, precision=HIGH)[:, 0]

    h = jnp.pad(h, ((0, 0), (1, 1), (1, 1), (0, 0)))
    z = _shifted_conv(h, _w_3x3(W3), b3.reshape(1, 1, -1), _TAPS3,
                      56, 56, relu=False, precision=HIGH)[:, 0]

    D = z.shape[-1]
    zf = z.reshape(-1, D)
    q = _vq(zf, codebook)
    zq = q.reshape(B, 56, 56, D)

    w2 = jnp.transpose(jnp.flip(Wd1, axis=(2, 3)), (1, 0, 2, 3))
    hq = jnp.pad(zq, ((0, 0), (1, 1), (1, 1), (0, 0)))
    h2 = _shifted_conv(hq, _w_3x3(w2), bd1.reshape(1, 1, -1), _TAPS3,
                       56, 56, relu=True, precision=LOW)[:, 0]

    w2 = jnp.transpose(jnp.flip(Wd2, axis=(2, 3)), (1, 0, 2, 3))
    h2 = jnp.pad(h2, ((0, 0), (1, 1), (1, 1), (0, 0)))
    bii = jnp.broadcast_to(bd2.reshape(1, 1, -1), (4, 1, 32))
    h3 = _shifted_conv(h2, _w_t_phases(w2), bii, _TAPS2,
                       56, 56, relu=True, precision=LOW,
                       phased=True)
    h3 = h3.reshape(B, 2, 2, 56, 56, 32).transpose(0, 3, 1, 4, 2, 5)
    h3 = h3.reshape(B, 112, 112, 32)

    w2 = jnp.transpose(jnp.flip(Wd3, axis=(2, 3)), (1, 0, 2, 3))
    h3 = jnp.pad(h3, ((0, 0), (1, 1), (1, 1), (0, 0)))
    bii = jnp.broadcast_to(
        jnp.pad(bd3, (0, 5)).reshape(1, 1, 8), (4, 1, 8))
    h4 = _shifted_conv(h3, _w_t_phases(w2, opad=8), bii, _TAPS2,
                       112, 112, relu=False, precision=LOW,
                       phased=True)
    h4 = h4.reshape(B, 2, 2, 112, 112, 8).transpose(0, 3, 1, 4, 2, 5)
    h4 = h4[..., :3].reshape(B, 224, 224, 3)
    return h4.transpose(0, 3, 1, 2).astype(jnp.float32)


# speedup vs baseline: 1.2246x; 1.2246x over previous
import jax
import jax.numpy as jnp
from jax.experimental import pallas as pl
from jax.experimental.pallas import tpu as pltpu


_TAPS2 = [(a, b) for a in (0, 1) for b in (0, 1)]
_TAPS3 = [(a, b) for a in (0, 1, 2) for b in (0, 1, 2)]
_TAPS4 = [(a, b) for a in (0, 1, 2, 3) for b in (0, 1, 2, 3)]


def _pad1(v):
    return jnp.pad(v, ((1, 1), (1, 1), (0, 0)))


def _mm(x, w):
    return jax.lax.dot_general(x, w, (((1,), (0,)), ((), ())),
                               preferred_element_type=jnp.float32)


def _conv(v, w, bvec, taps, H, W, stride, relu):
    C = v.shape[-1]
    parts = []
    for (dy, dx) in taps:
        parts.append(jax.lax.slice(
            v, (dy, dx, 0),
            (dy + (H - 1) * stride + 1, dx + (W - 1) * stride + 1, C),
            (stride, stride, 1)))
    xc = jnp.concatenate(parts, axis=-1).reshape(H * W, len(taps) * C)
    acc = _mm(xc, w) + bvec
    if relu:
        acc = jnp.maximum(acc, 0.0)
    return acc.reshape(H, W, w.shape[1])


def _vq_chunk(zc, cbt, cb, cb2):
    scores = _mm(zc, cbt)
    d2 = cb2 - 2.0 * scores
    dmin = jnp.min(d2, axis=1, keepdims=True)
    iota = jax.lax.broadcasted_iota(jnp.int32, d2.shape, 1)
    masked = jnp.where(d2 <= dmin, iota, d2.shape[1])
    idx = jnp.min(masked, axis=1, keepdims=True)
    onehot = (iota == idx).astype(jnp.float32)
    return _mm(onehot, cb)


def _body(xs_ref, w1_ref, b1_ref, w2_ref, b2_ref, w3_ref, b3_ref,
          cbt_ref, cb_ref, wd1_ref, bd1_ref, wd2_ref, bd2_ref,
          wd3_ref, bd3_ref, o_ref, scr_ref):
    h = _conv(xs_ref[...], w1_ref[...], b1_ref[...], _TAPS2,
              112, 112, 1, True)
    scr_ref[1:113, 1:113, :] = h
    scr_ref[0:1, :, :] = jnp.zeros((1, 114, 32), jnp.float32)
    scr_ref[113:114, :, :] = jnp.zeros((1, 114, 32), jnp.float32)
    scr_ref[1:113, 0:1, :] = jnp.zeros((112, 1, 32), jnp.float32)
    scr_ref[1:113, 113:114, :] = jnp.zeros((112, 1, 32), jnp.float32)
    parts = [scr_ref[pl.Slice(dy, 56, 2), pl.Slice(dx, 56, 2), :]
             for (dy, dx) in _TAPS4]
    xc = jnp.concatenate(parts, axis=-1).reshape(56 * 56, 16 * 32)
    h = jnp.maximum(_mm(xc, w2_ref[...]) + b2_ref[...], 0.0)
    h = h.reshape(56, 56, 64)
    z = _conv(_pad1(h), w3_ref[...], b3_ref[...], _TAPS3,
              56, 56, 1, False)

    zf = z.reshape(3136, 64)
    cbt = cbt_ref[...]
    cb = cb_ref[...]
    cb2 = jnp.sum(cbt * cbt, axis=0, keepdims=True)
    qs = []
    m = 784
    for ci in range(4):
        zc = jax.lax.slice(zf, (ci * m, 0), ((ci + 1) * m, 64))
        qs.append(_vq_chunk(zc, cbt, cb, cb2))
    q3 = jnp.concatenate(qs, axis=0).reshape(56, 56, 64)

    h = _conv(_pad1(q3), wd1_ref[...], bd1_ref[...], _TAPS3,
              56, 56, 1, True)

    hp = _pad1(h)
    for p, (r, s) in enumerate([(0, 0), (0, 1), (1, 0), (1, 1)]):
        acc = _conv(jax.lax.slice(hp, (r, s, 0), (r + 57, s + 57, 64)),
                    wd2_ref[p], bd2_ref[p], _TAPS2, 56, 56, 1, True)
        scr_ref[pl.Slice(1 + r, 56, 2), pl.Slice(1 + s, 56, 2), :] = acc

    hp = scr_ref[...]
    for p, (r, s) in enumerate([(0, 0), (0, 1), (1, 0), (1, 1)]):
        acc = _conv(jax.lax.slice(hp, (r, s, 0), (r + 113, s + 113, 32)),
                    wd3_ref[p], bd3_ref[p], _TAPS2, 112, 112, 1, False)
        o_ref[:, :, p * 8:(p + 1) * 8] = acc


def _w_s2d(w):
    O, C = w.shape[0], w.shape[1]
    wt = w.transpose(2, 3, 1, 0)
    wt = wt.reshape(2, 2, 2, 2, C, O)
    wt = wt.transpose(0, 2, 1, 3, 4, 5)
    return wt.reshape(4 * 4 * C, O)


def _w_taps(w):
    O, C, kh, kw = w.shape
    return w.transpose(2, 3, 1, 0).reshape(kh * kw * C, O)


def _w_t_phases(w, opad=None):
    O, I = w.shape[0], w.shape[1]
    wt = w.transpose(2, 3, 1, 0)
    if opad is not None and opad > O:
        wt = jnp.pad(wt, ((0, 0), (0, 0), (0, 0), (0, opad - O)))
        O = opad
    phases = []
    for r in (0, 1):
        for s in (0, 1):
            ph = wt[r::2, s::2]
            phases.append(ph.reshape(4 * I, O))
    return jnp.stack(phases)


def kernel(x, W1, b1, W2, b2, W3, b3, codebook, Wd1, bd1, Wd2, bd2, Wd3, bd3):
    B = x.shape[0]

    xh = x.transpose(0, 2, 3, 1)
    xh = jnp.pad(xh, ((0, 0), (1, 1), (1, 1), (0, 0)))
    xh = xh.reshape(B, 113, 2, 113, 2, 3).transpose(0, 1, 3, 2, 4, 5)
    xh = xh.reshape(B, 113, 113, 12)

    wd1e = jnp.transpose(jnp.flip(Wd1, axis=(2, 3)), (1, 0, 2, 3))
    wd2e = jnp.transpose(jnp.flip(Wd2, axis=(2, 3)), (1, 0, 2, 3))
    wd3e = jnp.transpose(jnp.flip(Wd3, axis=(2, 3)), (1, 0, 2, 3))

    args = (
        xh,
        _w_s2d(W1), b1.reshape(1, -1),
        _w_taps(W2), b2.reshape(1, -1),
        _w_taps(W3), b3.reshape(1, -1),
        codebook.T, codebook,
        _w_taps(wd1e), bd1.reshape(1, -1),
        _w_t_phases(wd2e), jnp.broadcast_to(bd2.reshape(1, 1, -1), (4, 1, 32)),
        _w_t_phases(wd3e, opad=8),
        jnp.broadcast_to(jnp.pad(bd3, (0, 5)).reshape(1, 1, 8), (4, 1, 8)),
    )

    def full(a):
        n = len(a.shape)
        return pl.BlockSpec(a.shape, lambda b, _n=n: (0,) * _n)

    in_specs = [pl.BlockSpec((None, 113, 113, 12), lambda b: (b, 0, 0, 0))]
    in_specs += [full(a) for a in args[1:]]

    out = pl.pallas_call(
        _body,
        grid=(B,),
        in_specs=in_specs,
        out_specs=pl.BlockSpec((None, 112, 112, 32), lambda b: (b, 0, 0, 0)),
        out_shape=jax.ShapeDtypeStruct((B, 112, 112, 32), jnp.float32),
        scratch_shapes=[pltpu.VMEM((114, 114, 32), jnp.float32)],
    )(*args)

    out = out.reshape(B, 112, 112, 2, 2, 8).transpose(0, 1, 3, 2, 4, 5)
    out = out.reshape(B, 224, 224, 8)[..., :3]
    return out.transpose(0, 3, 1, 2)
